# Initial kernel scaffold; baseline (speedup 1.0000x reference)
import jax
import jax.numpy as jnp
from jax.experimental import pallas as pl


def kernel(x, edge_index, edge_attr, u, ew1, eb1, ew2, eb2, n1w1, n1b1, n1w2, n1b2, n2w1, n2b1, n2w2, n2b2, gw1, gb1, gw2, gb2):
    raise NotImplementedError("write your pallas kernel here")


# speedup vs baseline: 2.3931x; 2.3931x over previous
import functools

import jax
import jax.numpy as jnp
from jax import lax
from jax.experimental import pallas as pl
from jax.experimental.pallas import tpu as pltpu
from jax.experimental.pallas import tpu_sc as plsc

N = 10000
E = 320000
DF = 128
DE = 16
DU = 32
H = 128

NC = 2
NS = 16
NW = NC * NS
CHUNK = 80
EPW = E // NW
NCHUNK = EPW // CHUNK
NPAD = 10240
RPT = NPAD // NS
CW = 16

BN = 1000
BE = 1280

_f32 = jnp.float32


def _prep_body(x_ref, wa_ref, wb_ref, wn_ref, u_ref, wu_ref, ba_ref, bn_ref,
               pa_ref, pb_ref, pn_ref):
    xb = x_ref[...]
    pa_ref[...] = jnp.dot(xb, wa_ref[...], preferred_element_type=_f32)
    ub = jnp.dot(u_ref[...], wu_ref[...], preferred_element_type=_f32)
    pb_ref[...] = jnp.dot(xb, wb_ref[...], preferred_element_type=_f32) + ba_ref[...] + ub
    pn_ref[...] = jnp.dot(xb, wn_ref[...], preferred_element_type=_f32) + bn_ref[...]


def _gather_body(pa_hbm, pb_hbm, pn_hbm, row_hbm, col_hbm,
                 ga_hbm, gb_hbm, gn_hbm,
                 idxr, idxc, bufa, bufb, bufn, sem):
    c = lax.axis_index("c")
    s = lax.axis_index("s")
    wid = c * NS + s

    def chunk(i, carry):
        base = wid * EPW + i * CHUNK
        pltpu.sync_copy(row_hbm.at[pl.ds(base, CHUNK)], idxr)
        pltpu.sync_copy(col_hbm.at[pl.ds(base, CHUNK)], idxc)
        cpa = pltpu.async_copy(pa_hbm.at[idxr], bufa, sem)
        cpb = pltpu.async_copy(pb_hbm.at[idxc], bufb, sem)
        cpn = pltpu.async_copy(pn_hbm.at[idxr], bufn, sem)
        cpa.wait()
        cpb.wait()
        cpn.wait()
        pltpu.sync_copy(bufa, ga_hbm.at[pl.ds(base, CHUNK)])
        pltpu.sync_copy(bufb, gb_hbm.at[pl.ds(base, CHUNK)])
        pltpu.sync_copy(bufn, gn_hbm.at[pl.ds(base, CHUNK)])
        return carry

    lax.fori_loop(0, NCHUNK, chunk, 0)


def _edge_body(ga_ref, gb_ref, gn_ref, attr_ref, wc_ref, ew2_ref, eb2_ref,
               n1wb_ref, eo_ref, h1_ref):
    he = jnp.maximum(
        ga_ref[...] + gb_ref[...]
        + jnp.dot(attr_ref[...], wc_ref[...], preferred_element_type=_f32),
        0.0)
    eo = jnp.dot(he, ew2_ref[...], preferred_element_type=_f32) + eb2_ref[...]
    eo_ref[...] = eo
    t = jnp.dot(eo, n1wb_ref[...], preferred_element_type=_f32)
    h1_ref[...] = jnp.maximum(gn_ref[...] + t, 0.0)


def _scatter_body(h1_hbm, col_hbm, zsum_hbm, zrow_hbm,
                  sum_hbm, cnt_hbm,
                  idxc, buf, bufc, acc, acc_cnt):
    c = lax.axis_index("c")
    s = lax.axis_index("s")
    pltpu.sync_copy(zsum_hbm, acc.at[pl.ds(s * RPT, RPT)])
    pltpu.sync_copy(zrow_hbm, acc_cnt.at[pl.ds(s * RPT, RPT)])
    ones16 = jnp.full((16,), 1.0, _f32)

    def fill(j, carry):
        bufc[pl.ds(j * 16, 16)] = ones16
        return carry

    lax.fori_loop(0, CHUNK // 16, fill, 0)
    plsc.subcore_barrier()

    def chunk(i, carry):
        base = (c * NS + s) * EPW + i * CHUNK
        pltpu.sync_copy(col_hbm.at[pl.ds(base, CHUNK)], idxc)
        pltpu.sync_copy(h1_hbm.at[pl.ds(base, CHUNK)], buf)
        pltpu.sync_copy(buf, acc.at[idxc], add=True)
        pltpu.sync_copy(bufc, acc_cnt.at[idxc], add=True)
        return carry

    lax.fori_loop(0, NCHUNK, chunk, 0)
    plsc.subcore_barrier()
    pltpu.sync_copy(acc.at[pl.ds(s * RPT, RPT)], sum_hbm.at[c, pl.ds(s * RPT, RPT)])
    pltpu.sync_copy(acc_cnt.at[pl.ds(s * RPT, RPT)], cnt_hbm.at[c, pl.ds(s * RPT, RPT)])


def _node_body(x_ref, sums_ref, cnts_ref, n1w2_ref, n1b2_ref, u_ref,
               w2a_ref, w2b_ref, w2c_ref, n2b1_ref, n2w2_ref, n2b2_ref,
               nout_ref, nsum_ref):
    S = sums_ref[0] + sums_ref[1]
    cnt = cnts_ref[0] + cnts_ref[1]
    cmax = jnp.maximum(cnt, 1.0)
    pos = (cnt > 0.0).astype(_f32)
    agg = jnp.dot(S, n1w2_ref[...], preferred_element_type=_f32) / cmax \
        + pos * n1b2_ref[...]
    ug = jnp.dot(u_ref[...], w2c_ref[...], preferred_element_type=_f32)
    h = jnp.maximum(
        jnp.dot(x_ref[...], w2a_ref[...], preferred_element_type=_f32)
        + jnp.dot(agg, w2b_ref[...], preferred_element_type=_f32)
        + ug + n2b1_ref[...],
        0.0)
    nout = jnp.dot(h, n2w2_ref[...], preferred_element_type=_f32) + n2b2_ref[...]
    nout_ref[...] = nout

    @pl.when(pl.program_id(0) == 0)
    def _():
        nsum_ref[...] = jnp.zeros_like(nsum_ref)

    nsum_ref[...] += jnp.sum(nout, axis=0, keepdims=True)


def _global_body(u_ref, nsum_ref, gu_ref, gm_ref, gb1_ref, gw2_ref, gb2_ref,
                 gout_ref):
    gmean = nsum_ref[...] * (1.0 / N)
    gh = jnp.maximum(
        jnp.dot(u_ref[...], gu_ref[...], preferred_element_type=_f32)
        + jnp.dot(gmean, gm_ref[...], preferred_element_type=_f32)
        + gb1_ref[...],
        0.0)
    gout_ref[...] = jnp.dot(gh, gw2_ref[...], preferred_element_type=_f32) + gb2_ref[...]


def _full(shape):
    return pl.BlockSpec(shape, lambda i: tuple(0 for _ in shape))


@functools.lru_cache(maxsize=None)
def _sc_calls():
    mesh = plsc.VectorSubcoreMesh(
        core_axis_name="c", subcore_axis_name="s",
        num_cores=NC, num_subcores=NS)
    gather = pl.kernel(
        _gather_body,
        out_type=[jax.ShapeDtypeStruct((E, DF), _f32)] * 3,
        mesh=mesh,
        scratch_types=[
            pltpu.VMEM((CHUNK,), jnp.int32),
            pltpu.VMEM((CHUNK,), jnp.int32),
            pltpu.VMEM((CHUNK, DF), _f32),
            pltpu.VMEM((CHUNK, DF), _f32),
            pltpu.VMEM((CHUNK, DF), _f32),
            pltpu.SemaphoreType.DMA,
        ],
    )
    scatter = pl.kernel(
        _scatter_body,
        out_type=[jax.ShapeDtypeStruct((NC, NPAD, H), _f32),
                  jax.ShapeDtypeStruct((NC, NPAD), _f32)],
        mesh=mesh,
        scratch_types=[
            pltpu.VMEM((CHUNK,), jnp.int32),
            pltpu.VMEM((CHUNK, H), _f32),
            pltpu.VMEM((CHUNK,), _f32),
            pltpu.VMEM_SHARED((NPAD, H), _f32),
            pltpu.VMEM_SHARED((NPAD,), _f32),
        ],
    )
    return gather, scatter


def kernel(x, edge_index, edge_attr, u,
           ew1, eb1, ew2, eb2,
           n1w1, n1b1, n1w2, n1b2,
           n2w1, n2b1, n2w2, n2b2,
           gw1, gb1, gw2, gb2):
    row = edge_index[0]
    col = edge_index[1]

    ew1_a = ew1[0:DF]
    ew1_b = ew1[DF:2 * DF]
    ew1_c = ew1[2 * DF:2 * DF + DE]
    ew1_u = ew1[2 * DF + DE:]
    n1w1_a = n1w1[0:DF]
    n1w1_b = n1w1[DF:DF + DE]
    n2w1_a = n2w1[0:DF]
    n2w1_b = n2w1[DF:2 * DF]
    n2w1_c = n2w1[2 * DF:]
    gw1_u = gw1[0:DU]
    gw1_m = gw1[DU:]

    eb1r = eb1.reshape(1, H)
    eb2r = eb2.reshape(1, DE)
    n1b1r = n1b1.reshape(1, DF)
    n1b2r = n1b2.reshape(1, DF)
    n2b1r = n2b1.reshape(1, H)
    n2b2r = n2b2.reshape(1, DF)
    gb1r = gb1.reshape(1, DU)
    gb2r = gb2.reshape(1, DU)

    pa, pb, pn = pl.pallas_call(
        _prep_body,
        grid=(N // BN,),
        in_specs=[
            pl.BlockSpec((BN, DF), lambda i: (i, 0)),
            _full((DF, H)), _full((DF, H)), _full((DF, H)),
            _full((1, DU)), _full((DU, H)), _full((1, H)), _full((1, H)),
        ],
        out_specs=[pl.BlockSpec((BN, H), lambda i: (i, 0))] * 3,
        out_shape=[jax.ShapeDtypeStruct((N, H), _f32)] * 3,
    )(x, ew1_a, ew1_b, n1w1_a, u, ew1_u, eb1r, n1b1r)

    _gather_call, _scatter_call = _sc_calls()
    ga, gb, gn = _gather_call(pa, pb, pn, row, col)

    eo, h1 = pl.pallas_call(
        _edge_body,
        grid=(E // BE,),
        in_specs=[
            pl.BlockSpec((BE, H), lambda i: (i, 0)),
            pl.BlockSpec((BE, H), lambda i: (i, 0)),
            pl.BlockSpec((BE, H), lambda i: (i, 0)),
            pl.BlockSpec((BE, DE), lambda i: (i, 0)),
            _full((DE, H)), _full((H, DE)), _full((1, DE)), _full((DE, H)),
        ],
        out_specs=[pl.BlockSpec((BE, DE), lambda i: (i, 0)),
                   pl.BlockSpec((BE, H), lambda i: (i, 0))],
        out_shape=[jax.ShapeDtypeStruct((E, DE), _f32),
                   jax.ShapeDtypeStruct((E, H), _f32)],
    )(ga, gb, gn, edge_attr, ew1_c, ew2, eb2r, n1w1_b)

    zsum = jnp.zeros((RPT, H), _f32)
    zrow = jnp.zeros((RPT,), _f32)
    sums, cnts = _scatter_call(h1, col, zsum, zrow)
    cnts3 = cnts[:, :N, None]

    nout, nsum = pl.pallas_call(
        _node_body,
        grid=(N // BN,),
        in_specs=[
            pl.BlockSpec((BN, DF), lambda i: (i, 0)),
            pl.BlockSpec((NC, BN, H), lambda i: (0, i, 0)),
            pl.BlockSpec((NC, BN, 1), lambda i: (0, i, 0)),
            _full((H, DF)), _full((1, DF)), _full((1, DU)),
            _full((DF, H)), _full((DF, H)), _full((DU, H)),
            _full((1, H)), _full((H, DF)), _full((1, DF)),
        ],
        out_specs=[pl.BlockSpec((BN, DF), lambda i: (i, 0)),
                   pl.BlockSpec((1, DF), lambda i: (0, 0))],
        out_shape=[jax.ShapeDtypeStruct((N, DF), _f32),
                   jax.ShapeDtypeStruct((1, DF), _f32)],
    )(x, sums, cnts3, n1w2, n1b2r, u, n2w1_a, n2w1_b, n2w1_c, n2b1r, n2w2, n2b2r)

    gout = pl.pallas_call(
        _global_body,
        grid=(1,),
        in_specs=[_full((1, DU)), _full((1, DF)), _full((DU, DU)),
                  _full((DF, DU)), _full((1, DU)), _full((DU, DU)), _full((1, DU))],
        out_specs=_full((1, DU)),
        out_shape=jax.ShapeDtypeStruct((1, DU), _f32),
    )(u, nsum, gw1_u, gw1_m, gb1r, gw2, gb2r)

    return nout, eo, gout
